# baseline (speedup 1.0000x reference)
import functools

import jax
import jax.numpy as jnp
from jax import lax
from jax.experimental import pallas as pl
from jax.experimental.pallas import tpu as pltpu
from jax.experimental.pallas import tpu_sc as plsc

TB = 4096

NC = 2
NS = 16
LANES = 16
NW = NC * NS


def _router_body(x_ref, w_ref, idx_ref, val_ref, hard1_ref, imp_ref, imp_acc):
    step = pl.program_id(0)
    nsteps = pl.num_programs(0)

    x = x_ref[...]
    w = w_ref[...]
    logits_tm = jnp.dot(x, w, preferred_element_type=jnp.float32)
    l = logits_tm.T

    m = jnp.max(l, axis=0, keepdims=True)
    i1 = jnp.argmax(l, axis=0).astype(jnp.int32)
    e = jnp.exp(l - m)
    s = jnp.sum(e, axis=0, keepdims=True)
    inv_s = 1.0 / s
    probs = e * inv_s

    rows = jax.lax.broadcasted_iota(jnp.int32, l.shape, 0)
    hit1 = rows == i1[None, :]
    masked = jnp.where(hit1, -jnp.inf, l)
    m2 = jnp.max(masked, axis=0, keepdims=True)
    i2 = jnp.argmax(masked, axis=0).astype(jnp.int32)
    v1 = inv_s
    v2 = jnp.exp(m2 - m) * inv_s

    idx_ref[0:1, :] = i1[None, :]
    idx_ref[1:2, :] = i2[None, :]
    val_ref[0:1, :] = v1
    val_ref[1:2, :] = v2
    hard1_ref[...] = i1

    blk_imp = jnp.sum(probs, axis=1, keepdims=True)

    @pl.when(step == 0)
    def _init():
        imp_acc[...] = blk_imp

    @pl.when(step != 0)
    def _accum():
        imp_acc[...] += blk_imp

    @pl.when(step == nsteps - 1)
    def _emit():
        imp_ref[...] = imp_acc[...].T


def _make_aux_sc(S, E):
    tpw = S // NW
    nch = tpw // LANES
    scale = float(E) / (float(S) * float(S))
    mesh = plsc.VectorSubcoreMesh(
        core_axis_name="c", subcore_axis_name="s",
        num_cores=NC, num_subcores=NS,
    )

    @functools.partial(
        pl.kernel,
        out_type=jax.ShapeDtypeStruct((NW, LANES), jnp.float32),
        mesh=mesh,
        scratch_types=[
            pltpu.VMEM((tpw,), jnp.int32),
            pltpu.VMEM((128,), jnp.float32),
            pltpu.VMEM((LANES,), jnp.float32),
        ],
        compiler_params=pltpu.CompilerParams(needs_layout_passes=False),
    )
    def _aux_body(hard1_hbm, imp_hbm, out_hbm, idx_v, imp_v, acc_v):
        cid = lax.axis_index("c")
        sid = lax.axis_index("s")
        wid = sid * NC + cid
        pltpu.sync_copy(hard1_hbm.at[pl.ds(wid * tpw, tpw)], idx_v)
        pltpu.sync_copy(imp_hbm, imp_v.at[pl.ds(0, E)])
        acc = jnp.zeros((LANES,), jnp.float32)
        for j in range(nch):
            idx = idx_v[pl.ds(j * LANES, LANES)]
            acc = acc + plsc.load_gather(imp_v, [idx])
        acc_v[...] = acc * scale
        pltpu.sync_copy(acc_v, out_hbm.at[wid])

    return _aux_body


def kernel(x, W):
    S, D = x.shape
    E = W.shape[1]
    grid = (S // TB,)

    idx_t, val_t, hard1, imp = pl.pallas_call(
        _router_body,
        grid=grid,
        in_specs=[
            pl.BlockSpec((TB, D), lambda i: (i, 0)),
            pl.BlockSpec((D, E), lambda i: (0, 0)),
        ],
        out_specs=[
            pl.BlockSpec((2, TB), lambda i: (0, i)),
            pl.BlockSpec((2, TB), lambda i: (0, i)),
            pl.BlockSpec((TB,), lambda i: (i,)),
            pl.BlockSpec((1, E), lambda i: (0, 0)),
        ],
        out_shape=[
            jax.ShapeDtypeStruct((2, S), jnp.int32),
            jax.ShapeDtypeStruct((2, S), jnp.float32),
            jax.ShapeDtypeStruct((S,), jnp.int32),
            jax.ShapeDtypeStruct((1, E), jnp.float32),
        ],
        scratch_shapes=[
            pltpu.VMEM((E, 1), jnp.float32),
        ],
        compiler_params=pltpu.CompilerParams(
            dimension_semantics=("arbitrary",),
        ),
    )(x, W)

    part = _make_aux_sc(S, E)(hard1, imp.reshape(E))
    return idx_t.T, val_t.T, jnp.sum(part)


# speedup vs baseline: 1.0029x; 1.0029x over previous
import functools

import jax
import jax.numpy as jnp
from jax import lax
from jax.experimental import pallas as pl
from jax.experimental.pallas import tpu as pltpu
from jax.experimental.pallas import tpu_sc as plsc

TB = 4096

NC = 2
NS = 16
LANES = 16
NW = NC * NS


def _router_body(x_ref, w_ref, idx_ref, val_ref, hard1_ref, imp_ref, imp_acc):
    step = pl.program_id(0)
    nsteps = pl.num_programs(0)

    x = x_ref[...]
    w = w_ref[...]
    logits_tm = jnp.dot(x, w, preferred_element_type=jnp.float32)
    l = logits_tm.T

    m = jnp.max(l, axis=0, keepdims=True)
    i1 = jnp.argmax(l, axis=0).astype(jnp.int32)
    e = jnp.exp(l - m)
    s = jnp.sum(e, axis=0, keepdims=True)
    inv_s = 1.0 / s
    probs = e * inv_s

    rows = jax.lax.broadcasted_iota(jnp.int32, l.shape, 0)
    hit1 = rows == i1[None, :]
    masked = jnp.where(hit1, -jnp.inf, l)
    m2 = jnp.max(masked, axis=0, keepdims=True)
    i2 = jnp.argmax(masked, axis=0).astype(jnp.int32)
    v1 = inv_s
    v2 = jnp.exp(m2 - m) * inv_s

    idx_ref[0:1, :] = i1[None, :]
    idx_ref[1:2, :] = i2[None, :]
    val_ref[0:1, :] = v1
    val_ref[1:2, :] = v2
    hard1_ref[...] = i1

    blk_imp = jnp.sum(probs, axis=1, keepdims=True)

    @pl.when(step == 0)
    def _init():
        imp_acc[...] = blk_imp

    @pl.when(step != 0)
    def _accum():
        imp_acc[...] += blk_imp

    @pl.when(step == nsteps - 1)
    def _emit():
        imp_ref[...] = imp_acc[...].T


def _make_aux_sc(S, E):
    tpw = S // NW
    nch = tpw // LANES
    scale = float(E) / (float(S) * float(S))
    mesh = plsc.VectorSubcoreMesh(
        core_axis_name="c", subcore_axis_name="s",
        num_cores=NC, num_subcores=NS,
    )

    @functools.partial(
        pl.kernel,
        out_type=jax.ShapeDtypeStruct((NW, LANES), jnp.float32),
        mesh=mesh,
        scratch_types=[
            pltpu.VMEM((tpw,), jnp.int32),
            pltpu.VMEM((128,), jnp.float32),
            pltpu.VMEM((LANES,), jnp.float32),
        ],
        compiler_params=pltpu.CompilerParams(
            needs_layout_passes=False, skip_device_barrier=True,
        ),
    )
    def _aux_body(hard1_hbm, imp_hbm, out_hbm, idx_v, imp_v, acc_v):
        cid = lax.axis_index("c")
        sid = lax.axis_index("s")
        wid = sid * NC + cid
        pltpu.sync_copy(hard1_hbm.at[pl.ds(wid * tpw, tpw)], idx_v)
        pltpu.sync_copy(imp_hbm, imp_v.at[pl.ds(0, E)])
        acc = jnp.zeros((LANES,), jnp.float32)
        for j in range(nch):
            idx = idx_v[pl.ds(j * LANES, LANES)]
            acc = acc + plsc.load_gather(imp_v, [idx])
        acc_v[...] = acc * scale
        pltpu.sync_copy(acc_v, out_hbm.at[wid])

    return _aux_body


def kernel(x, W):
    S, D = x.shape
    E = W.shape[1]
    grid = (S // TB,)

    idx_t, val_t, hard1, imp = pl.pallas_call(
        _router_body,
        grid=grid,
        in_specs=[
            pl.BlockSpec((TB, D), lambda i: (i, 0)),
            pl.BlockSpec((D, E), lambda i: (0, 0)),
        ],
        out_specs=[
            pl.BlockSpec((2, TB), lambda i: (0, i)),
            pl.BlockSpec((2, TB), lambda i: (0, i)),
            pl.BlockSpec((TB,), lambda i: (i,)),
            pl.BlockSpec((1, E), lambda i: (0, 0)),
        ],
        out_shape=[
            jax.ShapeDtypeStruct((2, S), jnp.int32),
            jax.ShapeDtypeStruct((2, S), jnp.float32),
            jax.ShapeDtypeStruct((S,), jnp.int32),
            jax.ShapeDtypeStruct((1, E), jnp.float32),
        ],
        scratch_shapes=[
            pltpu.VMEM((E, 1), jnp.float32),
        ],
        compiler_params=pltpu.CompilerParams(
            dimension_semantics=("arbitrary",),
        ),
    )(x, W)

    part = _make_aux_sc(S, E)(hard1, imp.reshape(E))
    return idx_t.T, val_t.T, jnp.sum(part)


# speedup vs baseline: 1.0279x; 1.0249x over previous
import functools

import jax
import jax.numpy as jnp
from jax import lax
from jax.experimental import pallas as pl
from jax.experimental.pallas import tpu as pltpu
from jax.experimental.pallas import tpu_sc as plsc

TB = 4096

NC = 2
NS = 16
LANES = 16
NW = NC * NS


def _router_body(x_ref, w_ref, idx_ref, val_ref, hard1_ref, imp_ref, imp_acc):
    step = pl.program_id(0)
    nsteps = pl.num_programs(0)

    x = x_ref[...]
    w = w_ref[...]
    logits_tm = jnp.dot(x, w, preferred_element_type=jnp.float32)
    l = logits_tm.T

    m = jnp.max(l, axis=0, keepdims=True)
    i1 = jnp.argmax(l, axis=0).astype(jnp.int32)
    e = jnp.exp(l - m)
    s = jnp.sum(e, axis=0, keepdims=True)
    inv_s = 1.0 / s
    probs = e * inv_s

    rows = jax.lax.broadcasted_iota(jnp.int32, l.shape, 0)
    hit1 = rows == i1[None, :]
    masked = jnp.where(hit1, -jnp.inf, l)
    m2 = jnp.max(masked, axis=0, keepdims=True)
    i2 = jnp.argmax(masked, axis=0).astype(jnp.int32)
    v1 = inv_s
    v2 = jnp.exp(m2 - m) * inv_s

    idx_ref[0:1, :] = i1[None, :]
    idx_ref[1:2, :] = i2[None, :]
    val_ref[0:1, :] = v1
    val_ref[1:2, :] = v2
    hard1_ref[...] = i1

    blk_imp = jnp.sum(probs, axis=1, keepdims=True)

    @pl.when(step == 0)
    def _init():
        imp_acc[...] = blk_imp

    @pl.when(step != 0)
    def _accum():
        imp_acc[...] += blk_imp

    @pl.when(step == nsteps - 1)
    def _emit():
        imp_ref[...] = imp_acc[...].T


def _make_aux_sc(S, E):
    tpw = S // NS
    nch = tpw // LANES
    scale = float(E) / (float(S) * float(S))
    mesh = plsc.VectorSubcoreMesh(
        core_axis_name="c", subcore_axis_name="s",
        num_cores=1, num_subcores=NS,
    )

    @functools.partial(
        pl.kernel,
        out_type=jax.ShapeDtypeStruct((NS, LANES), jnp.float32),
        mesh=mesh,
        scratch_types=[
            pltpu.VMEM((tpw,), jnp.int32),
            pltpu.VMEM((128,), jnp.float32),
            pltpu.VMEM((LANES,), jnp.float32),
        ],
        compiler_params=pltpu.CompilerParams(needs_layout_passes=False),
    )
    def _aux_body(hard1_hbm, imp_hbm, out_hbm, idx_v, imp_v, acc_v):
        sid = lax.axis_index("s")
        pltpu.sync_copy(hard1_hbm.at[pl.ds(sid * tpw, tpw)], idx_v)
        pltpu.sync_copy(imp_hbm, imp_v.at[pl.ds(0, E)])
        acc = jnp.zeros((LANES,), jnp.float32)
        for j in range(nch):
            idx = idx_v[pl.ds(j * LANES, LANES)]
            acc = acc + plsc.load_gather(imp_v, [idx])
        acc_v[...] = acc * scale
        pltpu.sync_copy(acc_v, out_hbm.at[sid])

    return _aux_body


def kernel(x, W):
    S, D = x.shape
    E = W.shape[1]
    grid = (S // TB,)

    idx_t, val_t, hard1, imp = pl.pallas_call(
        _router_body,
        grid=grid,
        in_specs=[
            pl.BlockSpec((TB, D), lambda i: (i, 0)),
            pl.BlockSpec((D, E), lambda i: (0, 0)),
        ],
        out_specs=[
            pl.BlockSpec((2, TB), lambda i: (0, i)),
            pl.BlockSpec((2, TB), lambda i: (0, i)),
            pl.BlockSpec((TB,), lambda i: (i,)),
            pl.BlockSpec((1, E), lambda i: (0, 0)),
        ],
        out_shape=[
            jax.ShapeDtypeStruct((2, S), jnp.int32),
            jax.ShapeDtypeStruct((2, S), jnp.float32),
            jax.ShapeDtypeStruct((S,), jnp.int32),
            jax.ShapeDtypeStruct((1, E), jnp.float32),
        ],
        scratch_shapes=[
            pltpu.VMEM((E, 1), jnp.float32),
        ],
        compiler_params=pltpu.CompilerParams(
            dimension_semantics=("arbitrary",),
        ),
    )(x, W)

    part = _make_aux_sc(S, E)(hard1, imp.reshape(E))
    return idx_t.T, val_t.T, jnp.sum(part)


# speedup vs baseline: 1.0417x; 1.0134x over previous
import functools

import jax
import jax.numpy as jnp
from jax import lax
from jax.experimental import pallas as pl
from jax.experimental.pallas import tpu as pltpu
from jax.experimental.pallas import tpu_sc as plsc

TB = 4096

NS = 16
LANES = 16


def _router_body(x_ref, w_ref, idx_ref, val_ref, hard1_ref, imp_ref, imp_acc):
    step = pl.program_id(0)
    nsteps = pl.num_programs(0)

    x = x_ref[...]
    w = w_ref[...]
    logits_tm = jnp.dot(x, w, preferred_element_type=jnp.float32)
    l = logits_tm.T

    m = jnp.max(l, axis=0, keepdims=True)
    i1 = jnp.argmax(l, axis=0).astype(jnp.int32)
    e = jnp.exp(l - m)
    s = jnp.sum(e, axis=0, keepdims=True)
    inv_s = 1.0 / s
    probs = e * inv_s

    rows = jax.lax.broadcasted_iota(jnp.int32, l.shape, 0)
    hit1 = rows == i1[None, :]
    masked = jnp.where(hit1, -jnp.inf, l)
    m2 = jnp.max(masked, axis=0, keepdims=True)
    i2 = jnp.argmax(masked, axis=0).astype(jnp.int32)
    v1 = inv_s
    v2 = jnp.exp(m2 - m) * inv_s

    idx_ref[0:1, :] = i1[None, :]
    idx_ref[1:2, :] = i2[None, :]
    val_ref[0:1, :] = v1
    val_ref[1:2, :] = v2
    hard1_ref[...] = i1

    blk_imp = jnp.sum(probs, axis=1, keepdims=True)

    @pl.when(step == 0)
    def _init():
        imp_acc[...] = blk_imp

    @pl.when(step != 0)
    def _accum():
        imp_acc[...] += blk_imp

    @pl.when(step == nsteps - 1)
    def _emit():
        imp_ref[...] = imp_acc[...].T


def _make_aux_sc(S, E):
    tpw = S // NS
    nch = tpw // LANES
    scale = float(E) / (float(S) * float(S))
    mesh = plsc.VectorSubcoreMesh(
        core_axis_name="c", subcore_axis_name="s",
        num_cores=1, num_subcores=NS,
    )

    @functools.partial(
        pl.kernel,
        out_type=jax.ShapeDtypeStruct((NS, LANES), jnp.float32),
        mesh=mesh,
        scratch_types=[
            pltpu.VMEM((tpw,), jnp.int32),
            pltpu.VMEM((128,), jnp.float32),
            pltpu.VMEM((LANES,), jnp.float32),
        ],
        compiler_params=pltpu.CompilerParams(needs_layout_passes=False),
    )
    def _aux_body(hard1_hbm, imp_hbm, out_hbm, idx_v, imp_v, acc_v):
        sid = lax.axis_index("s")
        pltpu.sync_copy(hard1_hbm.at[pl.ds(sid * tpw, tpw)], idx_v)
        pltpu.sync_copy(imp_hbm, imp_v.at[pl.ds(0, E)])
        acc = jnp.zeros((LANES,), jnp.float32)
        for j in range(nch):
            idx = idx_v[pl.ds(j * LANES, LANES)]
            acc = acc + plsc.load_gather(imp_v, [idx])
        acc_v[...] = acc * scale
        pltpu.sync_copy(acc_v, out_hbm.at[sid])

    return _aux_body


def kernel(x, W):
    S, D = x.shape
    E = W.shape[1]
    grid = (S // TB,)

    idx_t, val_t, hard1, imp = pl.pallas_call(
        _router_body,
        grid=grid,
        in_specs=[
            pl.BlockSpec((TB, D), lambda i: (i, 0)),
            pl.BlockSpec((D, E), lambda i: (0, 0)),
        ],
        out_specs=[
            pl.BlockSpec((2, TB), lambda i: (0, i)),
            pl.BlockSpec((2, TB), lambda i: (0, i)),
            pl.BlockSpec((TB,), lambda i: (i,)),
            pl.BlockSpec((1, E), lambda i: (0, 0)),
        ],
        out_shape=[
            jax.ShapeDtypeStruct((2, S), jnp.int32),
            jax.ShapeDtypeStruct((2, S), jnp.float32),
            jax.ShapeDtypeStruct((S,), jnp.int32),
            jax.ShapeDtypeStruct((1, E), jnp.float32),
        ],
        scratch_shapes=[
            pltpu.VMEM((E, 1), jnp.float32),
        ],
        compiler_params=pltpu.CompilerParams(
            dimension_semantics=("arbitrary",),
        ),
    )(x, W)

    part = _make_aux_sc(S, E)(hard1, imp.reshape(E))
    return idx_t.T, val_t.T, jnp.sum(part)
